# baseline (speedup 1.0000x reference)
import jax
import jax.numpy as jnp
from jax import lax
from jax.experimental import pallas as pl
from jax.experimental.pallas import tpu as pltpu
from jax.experimental.pallas import tpu_sc as plsc

_H = 128
_B = 4
_S = 2048

_NC = 2
_NS = 16
_NW = _NC * _NS
_POS_PER_W = _S // _NW
_LANES = 16
_VECS_PER_ROW = _H // _LANES


def _tok_pos_embed_sc(ids_hbm, tok_hbm, pos_hbm, out_hbm,
                      idx_v, tok_v, pos_v,
                      sem_g0, sem_g0h, sem_g1, sem_g2, sem_g3, sem_pos,
                      sem_out):
  wid = lax.axis_index("s") * _NC + lax.axis_index("c")
  pos_base = wid * _POS_PER_W

  gather_sems = [sem_g0, sem_g1, sem_g2, sem_g3]
  id_copies = [
      pltpu.async_copy(ids_hbm.at[b, pl.ds(pos_base, _POS_PER_W)],
                       idx_v.at[b], gather_sems[b])
      for b in range(_B)
  ]

  pos_copy = pltpu.async_copy(
      pos_hbm.at[pl.ds(pos_base, _POS_PER_W)], pos_v, sem_pos)

  _HALF = _POS_PER_W // 2
  copies = []
  for b in range(_B):
    id_copies[b].wait()
    if b == 0:
      copies.append(pltpu.async_copy(
          tok_hbm.at[idx_v.at[0, pl.ds(0, _HALF)]],
          tok_v.at[0, pl.ds(0, _HALF)], sem_g0))
      copies.append(pltpu.async_copy(
          tok_hbm.at[idx_v.at[0, pl.ds(_HALF, _HALF)]],
          tok_v.at[0, pl.ds(_HALF, _HALF)], sem_g0h))
    else:
      copies.append(
          pltpu.async_copy(tok_hbm.at[idx_v.at[b]], tok_v.at[b],
                           gather_sems[b]))
  pos_copy.wait()

  def add_rows(b, lo, hi):
    @plsc.parallel_loop(lo, hi)
    def row_body(r, b=b):
      for j in range(_VECS_PER_ROW):
        sl = pl.ds(j * _LANES, _LANES)
        plsc.addupdate(tok_v.at[b, r, sl], pos_v[r, sl])

  copies[0].wait()
  add_rows(0, 0, _HALF)
  copies[1].wait()
  add_rows(0, _HALF, _POS_PER_W)
  pltpu.async_copy(
      tok_v.at[0], out_hbm.at[0, pl.ds(pos_base, _POS_PER_W)], sem_out)

  for b in range(1, _B):
    copies[b + 1].wait()

  def batch_body(b, carry):
    @plsc.parallel_loop(0, _POS_PER_W)
    def row_body(r):
      for j in range(_VECS_PER_ROW):
        sl = pl.ds(j * _LANES, _LANES)
        plsc.addupdate(tok_v.at[b, r, sl], pos_v[r, sl])

    pltpu.async_copy(
        tok_v.at[b], out_hbm.at[b, pl.ds(pos_base, _POS_PER_W)], sem_out)
    return carry

  lax.fori_loop(1, _B, batch_body, 0)

  for _ in range(_B):
    pltpu.make_async_copy(
        tok_hbm.at[pl.ds(0, _POS_PER_W)], tok_v.at[0], sem_out).wait()


def kernel(input_ids, tok_table, pos_table):
  b, s = input_ids.shape
  if input_ids.dtype != jnp.int32:
    input_ids = input_ids.astype(jnp.int32)

  mesh = plsc.VectorSubcoreMesh(
      core_axis_name="c", subcore_axis_name="s",
      num_cores=_NC, num_subcores=_NS,
  )
  run = pl.kernel(
      _tok_pos_embed_sc,
      out_type=jax.ShapeDtypeStruct((b, s, _H), jnp.float32),
      mesh=mesh,
      scratch_types=[
          pltpu.VMEM((_B, _POS_PER_W), jnp.int32),
          pltpu.VMEM((_B, _POS_PER_W, _H), jnp.float32),
          pltpu.VMEM((_POS_PER_W, _H), jnp.float32),
          pltpu.SemaphoreType.DMA,
          pltpu.SemaphoreType.DMA,
          pltpu.SemaphoreType.DMA,
          pltpu.SemaphoreType.DMA,
          pltpu.SemaphoreType.DMA,
          pltpu.SemaphoreType.DMA,
          pltpu.SemaphoreType.DMA,
      ],
  )
  return run(input_ids, tok_table, pos_table)


# speedup vs baseline: 1.0075x; 1.0075x over previous
import jax
import jax.numpy as jnp
from jax import lax
from jax.experimental import pallas as pl
from jax.experimental.pallas import tpu as pltpu
from jax.experimental.pallas import tpu_sc as plsc

_H = 128
_B = 4
_S = 2048

_NC = 2
_NS = 16
_NW = _NC * _NS
_POS_PER_W = _S // _NW
_LANES = 16
_VECS_PER_ROW = _H // _LANES


def _tok_pos_embed_sc(ids_hbm, tok_hbm, pos_hbm, out_hbm,
                      idx_v, tok_v, pos_v,
                      sem_g0, sem_g0h, sem_g1, sem_g2, sem_g3, sem_pos,
                      sem_out):
  wid = lax.axis_index("s") * _NC + lax.axis_index("c")
  pos_base = wid * _POS_PER_W

  gather_sems = [sem_g0, sem_g1, sem_g2, sem_g3]
  id_copies = [
      pltpu.async_copy(ids_hbm.at[b, pl.ds(pos_base, _POS_PER_W)],
                       idx_v.at[b], gather_sems[b])
      for b in range(_B)
  ]

  pos_copy = pltpu.async_copy(
      pos_hbm.at[pl.ds(pos_base, _POS_PER_W)], pos_v, sem_pos)

  _HALF = _POS_PER_W // 2
  copies = []
  for b in range(_B):
    id_copies[b].wait()
    if b == 0:
      copies.append(pltpu.async_copy(
          tok_hbm.at[idx_v.at[0, pl.ds(0, _HALF)]],
          tok_v.at[0, pl.ds(0, _HALF)], sem_g0))
      copies.append(pltpu.async_copy(
          tok_hbm.at[idx_v.at[0, pl.ds(_HALF, _HALF)]],
          tok_v.at[0, pl.ds(_HALF, _HALF)], sem_g0h))
    else:
      copies.append(
          pltpu.async_copy(tok_hbm.at[idx_v.at[b]], tok_v.at[b],
                           gather_sems[b]))
  pos_copy.wait()

  def add_rows(b, lo, hi):
    @plsc.parallel_loop(lo, hi)
    def row_body(r, b=b):
      for j in range(_VECS_PER_ROW):
        sl = pl.ds(j * _LANES, _LANES)
        plsc.addupdate(tok_v.at[b, r, sl], pos_v[r, sl])

  stores = []
  copies[0].wait()
  add_rows(0, 0, _HALF)
  copies[1].wait()
  add_rows(0, _HALF, _POS_PER_W)
  stores.append(pltpu.async_copy(
      tok_v.at[0], out_hbm.at[pl.ds(pos_base, _POS_PER_W)], sem_out))
  for b in range(1, _B):
    copies[b + 1].wait()
    add_rows(b, 0, _POS_PER_W)
    stores.append(pltpu.async_copy(
        tok_v.at[b], out_hbm.at[pl.ds(b * _S + pos_base, _POS_PER_W)],
        sem_out))

  for s in stores:
    s.wait()


def kernel(input_ids, tok_table, pos_table):
  b, s = input_ids.shape
  if input_ids.dtype != jnp.int32:
    input_ids = input_ids.astype(jnp.int32)

  mesh = plsc.VectorSubcoreMesh(
      core_axis_name="c", subcore_axis_name="s",
      num_cores=_NC, num_subcores=_NS,
  )
  run = pl.kernel(
      _tok_pos_embed_sc,
      out_type=jax.ShapeDtypeStruct((b * s, _H), jnp.float32),
      mesh=mesh,
      scratch_types=[
          pltpu.VMEM((_B, _POS_PER_W), jnp.int32),
          pltpu.VMEM((_B, _POS_PER_W, _H), jnp.float32),
          pltpu.VMEM((_POS_PER_W, _H), jnp.float32),
          pltpu.SemaphoreType.DMA,
          pltpu.SemaphoreType.DMA,
          pltpu.SemaphoreType.DMA,
          pltpu.SemaphoreType.DMA,
          pltpu.SemaphoreType.DMA,
          pltpu.SemaphoreType.DMA,
          pltpu.SemaphoreType.DMA,
      ],
  )
  out = run(input_ids, tok_table, pos_table)
  return out.reshape(b, s, _H)
